# baseline (speedup 1.0000x reference)
import jax
import jax.numpy as jnp
from jax import lax
from jax.experimental import pallas as pl
from jax.experimental.pallas import tpu as pltpu
from jax.experimental.pallas import tpu_sc as plsc

N_NODES = 10000
N_EDGES = 320000
D = 128
H = 128


_NB = 2000


def _proj_body(nf_ref, ws_ref, wd_ref, ps_ref, pd_ref):
    x = nf_ref[...]
    ps_ref[...] = jnp.dot(x, ws_ref[...], preferred_element_type=jnp.float32)
    pd_ref[...] = jnp.dot(x, wd_ref[...], preferred_element_type=jnp.float32)


def _project_nodes(nfeat, W_s, W_d):
    return pl.pallas_call(
        _proj_body,
        grid=(N_NODES // _NB,),
        in_specs=[
            pl.BlockSpec((_NB, D), lambda i: (i, 0)),
            pl.BlockSpec((D, H), lambda i: (0, 0)),
            pl.BlockSpec((D, H), lambda i: (0, 0)),
        ],
        out_specs=[
            pl.BlockSpec((_NB, H), lambda i: (i, 0)),
            pl.BlockSpec((_NB, H), lambda i: (i, 0)),
        ],
        out_shape=[
            jax.ShapeDtypeStruct((N_NODES, H), jnp.float32),
            jax.ShapeDtypeStruct((N_NODES, H), jnp.float32),
        ],
    )(nfeat, W_s, W_d)


_NC = 2
_NS = 16
_NW = _NC * _NS
_C = 128
_TPW = 80
_EPW = _TPW * _C
E_PAD = _NW * _EPW


def _gather_body(src_hbm, dst_hbm, ps_hbm, pd_hbm, out_hbm,
                 isrc0, idst0, ra0, rb0,
                 isrc1, idst1, ra1, rb1,
                 gsem0, gsem1, wsem0, wsem1):
    wid = lax.axis_index("s") * _NC + lax.axis_index("c")
    base = wid * _EPW
    slots = ((isrc0, idst0, ra0, rb0, gsem0, wsem0),
             (isrc1, idst1, ra1, rb1, gsem1, wsem1))

    def g_start(k, s):
        isrc, idst, ra, rb, gsem, _ = slots[s]
        off = base + k * _C
        pltpu.sync_copy(src_hbm.at[pl.ds(off, _C)], isrc)
        pltpu.sync_copy(dst_hbm.at[pl.ds(off, _C)], idst)
        pltpu.async_copy(ps_hbm.at[isrc], ra, gsem)
        pltpu.async_copy(pd_hbm.at[idst], rb, gsem)

    def g_wait(s):
        isrc, idst, ra, rb, gsem, _ = slots[s]
        pltpu.make_async_copy(ps_hbm.at[isrc], ra, gsem).wait()
        pltpu.make_async_copy(pd_hbm.at[idst], rb, gsem).wait()

    def add(s):
        _, _, ra, rb, _, _ = slots[s]

        def add_row(e, c2):
            for j in range(H // 16):
                sl = pl.ds(j * 16, 16)
                ra[e, sl] = ra[e, sl] + rb[e, sl]
            return c2

        lax.fori_loop(0, _C, add_row, 0)

    def wb_sync(k, s):
        _, _, ra, _, _, _ = slots[s]
        off = base + k * _C
        pltpu.sync_copy(ra, out_hbm.at[pl.ds(off, _C)])

    g_start(0, 0)

    def step(k2, carry):
        a = 2 * k2
        g_start(a + 1, 1)
        g_wait(0)
        add(0)
        wb_sync(a, 0)

        @pl.when(k2 < _TPW // 2 - 1)
        def _():
            g_start(a + 2, 0)

        g_wait(1)
        add(1)
        wb_sync(a + 1, 1)
        return carry

    lax.fori_loop(0, _TPW // 2, step, 0)


def _gather_sum(src, dst, ps, pd):
    mesh = plsc.VectorSubcoreMesh(core_axis_name="c", subcore_axis_name="s")
    f = pl.kernel(
        _gather_body,
        mesh=mesh,
        out_type=jax.ShapeDtypeStruct((E_PAD, H), jnp.float32),
        scratch_types=[
            pltpu.VMEM((_C,), jnp.int32),
            pltpu.VMEM((_C,), jnp.int32),
            pltpu.VMEM((_C, H), jnp.float32),
            pltpu.VMEM((_C, H), jnp.float32),
            pltpu.VMEM((_C,), jnp.int32),
            pltpu.VMEM((_C,), jnp.int32),
            pltpu.VMEM((_C, H), jnp.float32),
            pltpu.VMEM((_C, H), jnp.float32),
            pltpu.SemaphoreType.DMA,
            pltpu.SemaphoreType.DMA,
            pltpu.SemaphoreType.DMA,
            pltpu.SemaphoreType.DMA,
        ],
    )
    pad = E_PAD - N_EDGES
    fill = jnp.arange(pad, dtype=jnp.int32) % N_NODES
    src_p = jnp.concatenate([src, fill])
    dst_p = jnp.concatenate([dst, fill])
    return f(src_p, dst_p, ps, pd)


_EB = 2000


def _edge_body(ef_ref, g_ref, we_ref, wo_ref, b1_ref, bo_ref, gm_ref, bt_ref,
               out_ref):
    ef = ef_ref[...]
    pre = (jnp.dot(ef.astype(jnp.bfloat16), we_ref[...].astype(jnp.bfloat16),
                   preferred_element_type=jnp.float32)
           + g_ref[...] + b1_ref[...])
    h = pre * (1.0 / (1.0 + jnp.exp(-pre)))
    o = (jnp.dot(h.astype(jnp.bfloat16), wo_ref[...].astype(jnp.bfloat16),
                 preferred_element_type=jnp.float32) + bo_ref[...])
    mean = jnp.mean(o, axis=-1, keepdims=True)
    cent = o - mean
    var = jnp.mean(cent * cent, axis=-1, keepdims=True)
    out_ref[...] = (gm_ref[...] * cent * lax.rsqrt(var + 1e-5)
                    + bt_ref[...] + ef)


def _edge_mlp(efeat, gsum, W_e, W_out, b1, b_out, gamma, beta):
    vec = lambda: pl.BlockSpec((1, D), lambda i: (0, 0))
    return pl.pallas_call(
        _edge_body,
        grid=(N_EDGES // _EB,),
        in_specs=[
            pl.BlockSpec((_EB, D), lambda i: (i, 0)),
            pl.BlockSpec((_EB, H), lambda i: (i, 0)),
            pl.BlockSpec((D, H), lambda i: (0, 0)),
            pl.BlockSpec((H, D), lambda i: (0, 0)),
            vec(), vec(), vec(), vec(),
        ],
        out_specs=pl.BlockSpec((_EB, D), lambda i: (i, 0)),
        out_shape=jax.ShapeDtypeStruct((N_EDGES, D), jnp.float32),
    )(efeat, gsum, W_e, W_out,
      b1.reshape(1, D), b_out.reshape(1, D),
      gamma.reshape(1, D), beta.reshape(1, D))


def kernel(efeat, nfeat, edge_index, W_e, W_s, W_d, b1, W_out, b_out, gamma,
           beta):
    src = edge_index[0].astype(jnp.int32)
    dst = edge_index[1].astype(jnp.int32)
    ps, pd = _project_nodes(nfeat, W_s, W_d)
    gsum = _gather_sum(src, dst, ps, pd)
    out = _edge_mlp(efeat, gsum, W_e, W_out, b1, b_out, gamma, beta)
    return (out, nfeat)


# speedup vs baseline: 1.1344x; 1.1344x over previous
import jax
import jax.numpy as jnp
from jax import lax
from jax.experimental import pallas as pl
from jax.experimental.pallas import tpu as pltpu
from jax.experimental.pallas import tpu_sc as plsc

N_NODES = 10000
N_EDGES = 320000
D = 128
H = 128


_NB = 2000


def _proj_body(nf_ref, ws_ref, wd_ref, ps_ref, pd_ref):
    x = nf_ref[...]
    ps_ref[...] = jnp.dot(x, ws_ref[...], preferred_element_type=jnp.float32)
    pd_ref[...] = jnp.dot(x, wd_ref[...], preferred_element_type=jnp.float32)


def _project_nodes(nfeat, W_s, W_d):
    return pl.pallas_call(
        _proj_body,
        grid=(N_NODES // _NB,),
        in_specs=[
            pl.BlockSpec((_NB, D), lambda i: (i, 0)),
            pl.BlockSpec((D, H), lambda i: (0, 0)),
            pl.BlockSpec((D, H), lambda i: (0, 0)),
        ],
        out_specs=[
            pl.BlockSpec((_NB, H), lambda i: (i, 0)),
            pl.BlockSpec((_NB, H), lambda i: (i, 0)),
        ],
        out_shape=[
            jax.ShapeDtypeStruct((N_NODES, H), jnp.float32),
            jax.ShapeDtypeStruct((N_NODES, H), jnp.float32),
        ],
    )(nfeat, W_s, W_d)


_NC = 2
_NS = 16
_NW = _NC * _NS
_C = 128
_TPW = 80
_EPW = _TPW * _C
E_PAD = _NW * _EPW


def _gather_body(src_hbm, dst_hbm, ps_hbm, pd_hbm, out_hbm,
                 isrc0, idst0, ra0, rb0,
                 isrc1, idst1, ra1, rb1,
                 gsem0, gsem1, wsem0, wsem1):
    wid = lax.axis_index("s") * _NC + lax.axis_index("c")
    base = wid * _EPW
    slots = ((isrc0, idst0, ra0, rb0, gsem0, wsem0),
             (isrc1, idst1, ra1, rb1, gsem1, wsem1))

    def g_start(k, s):
        isrc, idst, ra, rb, gsem, _ = slots[s]
        off = base + k * _C
        pltpu.sync_copy(src_hbm.at[pl.ds(off, _C)], isrc)
        pltpu.sync_copy(dst_hbm.at[pl.ds(off, _C)], idst)
        pltpu.async_copy(ps_hbm.at[isrc], ra, gsem)
        pltpu.async_copy(pd_hbm.at[idst], rb, gsem)

    def g_wait(s):
        isrc, idst, ra, rb, gsem, _ = slots[s]
        pltpu.make_async_copy(ps_hbm.at[isrc], ra, gsem).wait()
        pltpu.make_async_copy(pd_hbm.at[idst], rb, gsem).wait()

    def add(s):
        _, _, ra, rb, _, _ = slots[s]

        def add_row(e, c2):
            for j in range(H // 16):
                sl = pl.ds(j * 16, 16)
                ra[e, sl] = ra[e, sl] + rb[e, sl]
            return c2

        lax.fori_loop(0, _C, add_row, 0)

    def wb_sync(k, s):
        _, _, ra, _, _, _ = slots[s]
        off = base + k * _C
        pltpu.sync_copy(ra, out_hbm.at[pl.ds(off, _C)])

    g_start(0, 0)

    def step(k2, carry):
        a = 2 * k2
        g_start(a + 1, 1)
        g_wait(0)
        add(0)
        wb_sync(a, 0)

        @pl.when(k2 < _TPW // 2 - 1)
        def _():
            g_start(a + 2, 0)

        g_wait(1)
        add(1)
        wb_sync(a + 1, 1)
        return carry

    lax.fori_loop(0, _TPW // 2, step, 0)


def _gather_sum(src, dst, ps, pd):
    mesh = plsc.VectorSubcoreMesh(core_axis_name="c", subcore_axis_name="s")
    f = pl.kernel(
        _gather_body,
        mesh=mesh,
        out_type=jax.ShapeDtypeStruct((E_PAD, H), jnp.float32),
        scratch_types=[
            pltpu.VMEM((_C,), jnp.int32),
            pltpu.VMEM((_C,), jnp.int32),
            pltpu.VMEM((_C, H), jnp.float32),
            pltpu.VMEM((_C, H), jnp.float32),
            pltpu.VMEM((_C,), jnp.int32),
            pltpu.VMEM((_C,), jnp.int32),
            pltpu.VMEM((_C, H), jnp.float32),
            pltpu.VMEM((_C, H), jnp.float32),
            pltpu.SemaphoreType.DMA,
            pltpu.SemaphoreType.DMA,
            pltpu.SemaphoreType.DMA,
            pltpu.SemaphoreType.DMA,
        ],
    )
    pad = E_PAD - N_EDGES
    fill = jnp.arange(pad, dtype=jnp.int32) % N_NODES
    src_p = jnp.concatenate([src, fill])
    dst_p = jnp.concatenate([dst, fill])
    return f(src_p, dst_p, ps, pd)


_EB = 4000


def _edge_body(ef_ref, g_ref, we_ref, wo_ref, b1_ref, bo_ref, gm_ref, bt_ref,
               out_ref):
    ef = ef_ref[...]
    pre = (jnp.dot(ef, we_ref[...], preferred_element_type=jnp.float32)
           + g_ref[...] + b1_ref[...])
    h = pre * (1.0 / (1.0 + jnp.exp(-pre)))
    o = jnp.dot(h, wo_ref[...], preferred_element_type=jnp.float32) + bo_ref[...]
    mean = jnp.mean(o, axis=-1, keepdims=True)
    cent = o - mean
    var = jnp.mean(cent * cent, axis=-1, keepdims=True)
    out_ref[...] = (gm_ref[...] * cent * lax.rsqrt(var + 1e-5)
                    + bt_ref[...] + ef)


def _edge_mlp(efeat, gsum, W_e, W_out, b1, b_out, gamma, beta):
    vec = lambda: pl.BlockSpec((1, D), lambda i: (0, 0))
    return pl.pallas_call(
        _edge_body,
        grid=(N_EDGES // _EB,),
        in_specs=[
            pl.BlockSpec((_EB, D), lambda i: (i, 0)),
            pl.BlockSpec((_EB, H), lambda i: (i, 0)),
            pl.BlockSpec((D, H), lambda i: (0, 0)),
            pl.BlockSpec((H, D), lambda i: (0, 0)),
            vec(), vec(), vec(), vec(),
        ],
        out_specs=pl.BlockSpec((_EB, D), lambda i: (i, 0)),
        out_shape=jax.ShapeDtypeStruct((N_EDGES, D), jnp.float32),
    )(efeat, gsum, W_e, W_out,
      b1.reshape(1, D), b_out.reshape(1, D),
      gamma.reshape(1, D), beta.reshape(1, D))


def kernel(efeat, nfeat, edge_index, W_e, W_s, W_d, b1, W_out, b_out, gamma,
           beta):
    src = edge_index[0].astype(jnp.int32)
    dst = edge_index[1].astype(jnp.int32)
    ps, pd = _project_nodes(nfeat, W_s, W_d)
    gsum = _gather_sum(src, dst, ps, pd)
    out = _edge_mlp(efeat, gsum, W_e, W_out, b1, b_out, gamma, beta)
    return (out, nfeat)


# speedup vs baseline: 1.1996x; 1.0575x over previous
import jax
import jax.numpy as jnp
from jax import lax
from jax.experimental import pallas as pl
from jax.experimental.pallas import tpu as pltpu
from jax.experimental.pallas import tpu_sc as plsc

N_NODES = 10000
N_EDGES = 320000
D = 128
H = 128


_NB = 2000


def _proj_body(nf_ref, ws_ref, wd_ref, ps_ref, pd_ref):
    x = nf_ref[...]
    ps_ref[...] = jnp.dot(x, ws_ref[...], preferred_element_type=jnp.float32)
    pd_ref[...] = jnp.dot(x, wd_ref[...], preferred_element_type=jnp.float32)


def _project_nodes(nfeat, W_s, W_d):
    return pl.pallas_call(
        _proj_body,
        grid=(N_NODES // _NB,),
        in_specs=[
            pl.BlockSpec((_NB, D), lambda i: (i, 0)),
            pl.BlockSpec((D, H), lambda i: (0, 0)),
            pl.BlockSpec((D, H), lambda i: (0, 0)),
        ],
        out_specs=[
            pl.BlockSpec((_NB, H), lambda i: (i, 0)),
            pl.BlockSpec((_NB, H), lambda i: (i, 0)),
        ],
        out_shape=[
            jax.ShapeDtypeStruct((N_NODES, H), jnp.float32),
            jax.ShapeDtypeStruct((N_NODES, H), jnp.float32),
        ],
    )(nfeat, W_s, W_d)


_NC = 2
_NS = 16
_NW = _NC * _NS
_C = 128
_TPW = 80
_EPW = _TPW * _C
E_PAD = _NW * _EPW


def _gather_body(src_hbm, dst_hbm, ps_hbm, pd_hbm, out_hbm,
                 isrc0, idst0, ra0, rb0,
                 isrc1, idst1, ra1, rb1,
                 gsem0, gsem1, wsem0, wsem1):
    wid = lax.axis_index("s") * _NC + lax.axis_index("c")
    base = wid * _EPW
    slots = ((isrc0, idst0, ra0, rb0, gsem0, wsem0),
             (isrc1, idst1, ra1, rb1, gsem1, wsem1))

    def g_start(k, s):
        isrc, idst, ra, rb, gsem, _ = slots[s]
        off = base + k * _C
        pltpu.sync_copy(src_hbm.at[pl.ds(off, _C)], isrc)
        pltpu.sync_copy(dst_hbm.at[pl.ds(off, _C)], idst)
        pltpu.async_copy(ps_hbm.at[isrc], ra, gsem)
        pltpu.async_copy(pd_hbm.at[idst], rb, gsem)

    def g_wait(s):
        isrc, idst, ra, rb, gsem, _ = slots[s]
        pltpu.make_async_copy(ps_hbm.at[isrc], ra, gsem).wait()
        pltpu.make_async_copy(pd_hbm.at[idst], rb, gsem).wait()

    def add(s):
        _, _, ra, rb, _, _ = slots[s]

        def add_row(e, c2):
            for j in range(H // 16):
                sl = pl.ds(j * 16, 16)
                ra[e, sl] = ra[e, sl] + rb[e, sl]
            return c2

        lax.fori_loop(0, _C, add_row, 0)

    def wb_sync(k, s):
        _, _, ra, _, _, _ = slots[s]
        off = base + k * _C
        pltpu.sync_copy(ra, out_hbm.at[pl.ds(off, _C)])

    g_start(0, 0)

    def step(k2, carry):
        a = 2 * k2
        g_start(a + 1, 1)
        g_wait(0)
        add(0)
        wb_sync(a, 0)

        @pl.when(k2 < _TPW // 2 - 1)
        def _():
            g_start(a + 2, 0)

        g_wait(1)
        add(1)
        wb_sync(a + 1, 1)
        return carry

    lax.fori_loop(0, _TPW // 2, step, 0)


def _gather_sum(src, dst, ps, pd):
    mesh = plsc.VectorSubcoreMesh(core_axis_name="c", subcore_axis_name="s")
    f = pl.kernel(
        _gather_body,
        mesh=mesh,
        out_type=jax.ShapeDtypeStruct((E_PAD, H), jnp.float32),
        scratch_types=[
            pltpu.VMEM((_C,), jnp.int32),
            pltpu.VMEM((_C,), jnp.int32),
            pltpu.VMEM((_C, H), jnp.float32),
            pltpu.VMEM((_C, H), jnp.float32),
            pltpu.VMEM((_C,), jnp.int32),
            pltpu.VMEM((_C,), jnp.int32),
            pltpu.VMEM((_C, H), jnp.float32),
            pltpu.VMEM((_C, H), jnp.float32),
            pltpu.SemaphoreType.DMA,
            pltpu.SemaphoreType.DMA,
            pltpu.SemaphoreType.DMA,
            pltpu.SemaphoreType.DMA,
        ],
    )
    pad = E_PAD - N_EDGES
    fill = jnp.arange(pad, dtype=jnp.int32) % N_NODES
    src_p = jnp.concatenate([src, fill])
    dst_p = jnp.concatenate([dst, fill])
    return f(src_p, dst_p, ps, pd)


_EB = 8000


def _edge_body(ef_ref, g_ref, we_ref, wo_ref, b1_ref, bo_ref, gm_ref, bt_ref,
               out_ref):
    ef = ef_ref[...]
    pre = (jnp.dot(ef, we_ref[...], preferred_element_type=jnp.float32)
           + g_ref[...] + b1_ref[...])
    h = pre * (1.0 / (1.0 + jnp.exp(-pre)))
    o = jnp.dot(h, wo_ref[...], preferred_element_type=jnp.float32) + bo_ref[...]
    mean = jnp.mean(o, axis=-1, keepdims=True)
    cent = o - mean
    var = jnp.mean(cent * cent, axis=-1, keepdims=True)
    out_ref[...] = (gm_ref[...] * cent * lax.rsqrt(var + 1e-5)
                    + bt_ref[...] + ef)


def _edge_mlp(efeat, gsum, W_e, W_out, b1, b_out, gamma, beta):
    vec = lambda: pl.BlockSpec((1, D), lambda i: (0, 0))
    return pl.pallas_call(
        _edge_body,
        grid=(N_EDGES // _EB,),
        in_specs=[
            pl.BlockSpec((_EB, D), lambda i: (i, 0)),
            pl.BlockSpec((_EB, H), lambda i: (i, 0)),
            pl.BlockSpec((D, H), lambda i: (0, 0)),
            pl.BlockSpec((H, D), lambda i: (0, 0)),
            vec(), vec(), vec(), vec(),
        ],
        out_specs=pl.BlockSpec((_EB, D), lambda i: (i, 0)),
        out_shape=jax.ShapeDtypeStruct((N_EDGES, D), jnp.float32),
    )(efeat, gsum, W_e, W_out,
      b1.reshape(1, D), b_out.reshape(1, D),
      gamma.reshape(1, D), beta.reshape(1, D))


def kernel(efeat, nfeat, edge_index, W_e, W_s, W_d, b1, W_out, b_out, gamma,
           beta):
    src = edge_index[0].astype(jnp.int32)
    dst = edge_index[1].astype(jnp.int32)
    ps, pd = _project_nodes(nfeat, W_s, W_d)
    gsum = _gather_sum(src, dst, ps, pd)
    out = _edge_mlp(efeat, gsum, W_e, W_out, b1, b_out, gamma, beta)
    return (out, nfeat)


# speedup vs baseline: 1.2431x; 1.0363x over previous
import jax
import jax.numpy as jnp
from jax import lax
from jax.experimental import pallas as pl
from jax.experimental.pallas import tpu as pltpu
from jax.experimental.pallas import tpu_sc as plsc

N_NODES = 10000
N_EDGES = 320000
D = 128
H = 128


_NB = 2000


def _proj_body(nf_ref, ws_ref, wd_ref, ps_ref, pd_ref):
    x = nf_ref[...]
    ps_ref[...] = jnp.dot(x, ws_ref[...], preferred_element_type=jnp.float32)
    pd_ref[...] = jnp.dot(x, wd_ref[...], preferred_element_type=jnp.float32)


def _project_nodes(nfeat, W_s, W_d):
    return pl.pallas_call(
        _proj_body,
        grid=(N_NODES // _NB,),
        in_specs=[
            pl.BlockSpec((_NB, D), lambda i: (i, 0)),
            pl.BlockSpec((D, H), lambda i: (0, 0)),
            pl.BlockSpec((D, H), lambda i: (0, 0)),
        ],
        out_specs=[
            pl.BlockSpec((_NB, H), lambda i: (i, 0)),
            pl.BlockSpec((_NB, H), lambda i: (i, 0)),
        ],
        out_shape=[
            jax.ShapeDtypeStruct((N_NODES, H), jnp.float32),
            jax.ShapeDtypeStruct((N_NODES, H), jnp.float32),
        ],
    )(nfeat, W_s, W_d)


_NC = 2
_NS = 16
_NW = _NC * _NS
_C = 128
_NHALF = 2
_TPW = 40
_EPW = _TPW * _C
_EH = _NW * _EPW
_EHR = N_EDGES // _NHALF


def _gather_body(src_hbm, dst_hbm, ps_hbm, pd_hbm, out_hbm,
                 isrc0, idst0, ra0, rb0,
                 isrc1, idst1, ra1, rb1,
                 gsem0, gsem1):
    wid = lax.axis_index("s") * _NC + lax.axis_index("c")
    base = wid * _EPW
    slots = ((isrc0, idst0, ra0, rb0, gsem0),
             (isrc1, idst1, ra1, rb1, gsem1))

    def g_start(k, s):
        isrc, idst, ra, rb, gsem = slots[s]
        off = base + k * _C
        pltpu.sync_copy(src_hbm.at[pl.ds(off, _C)], isrc)
        pltpu.sync_copy(dst_hbm.at[pl.ds(off, _C)], idst)
        pltpu.async_copy(ps_hbm.at[isrc], ra, gsem)
        pltpu.async_copy(pd_hbm.at[idst], rb, gsem)

    def g_wait(s):
        isrc, idst, ra, rb, gsem = slots[s]
        pltpu.make_async_copy(ps_hbm.at[isrc], ra, gsem).wait()
        pltpu.make_async_copy(pd_hbm.at[idst], rb, gsem).wait()

    def add(s):
        _, _, ra, rb, _ = slots[s]

        def add_row(e, c2):
            for j in range(H // 16):
                sl = pl.ds(j * 16, 16)
                ra[e, sl] = ra[e, sl] + rb[e, sl]
            return c2

        lax.fori_loop(0, _C, add_row, 0)

    def wb_sync(k, s):
        _, _, ra, _, _ = slots[s]
        off = base + k * _C
        pltpu.sync_copy(ra, out_hbm.at[pl.ds(off, _C)])

    g_start(0, 0)

    def step(k2, carry):
        a = 2 * k2
        g_start(a + 1, 1)
        g_wait(0)
        add(0)
        wb_sync(a, 0)

        @pl.when(k2 < _TPW // 2 - 1)
        def _():
            g_start(a + 2, 0)

        g_wait(1)
        add(1)
        wb_sync(a + 1, 1)
        return carry

    lax.fori_loop(0, _TPW // 2, step, 0)


def _gather_sum_half(src_h, dst_h, ps, pd):
    mesh = plsc.VectorSubcoreMesh(core_axis_name="c", subcore_axis_name="s")
    f = pl.kernel(
        _gather_body,
        mesh=mesh,
        out_type=jax.ShapeDtypeStruct((_EH, H), jnp.float32),
        scratch_types=[
            pltpu.VMEM((_C,), jnp.int32),
            pltpu.VMEM((_C,), jnp.int32),
            pltpu.VMEM((_C, H), jnp.float32),
            pltpu.VMEM((_C, H), jnp.float32),
            pltpu.VMEM((_C,), jnp.int32),
            pltpu.VMEM((_C,), jnp.int32),
            pltpu.VMEM((_C, H), jnp.float32),
            pltpu.VMEM((_C, H), jnp.float32),
            pltpu.SemaphoreType.DMA,
            pltpu.SemaphoreType.DMA,
        ],
    )
    return f(src_h, dst_h, ps, pd)


_EB = 8000
_NBLK = (N_EDGES // _NHALF) // _EB


def _edge_compute(ef, g, we, wo, b1, bo, gm, bt):
    pre = jnp.dot(ef, we, preferred_element_type=jnp.float32) + g + b1
    h = pre * (1.0 / (1.0 + jnp.exp(-pre)))
    o = jnp.dot(h, wo, preferred_element_type=jnp.float32) + bo
    mean = jnp.mean(o, axis=-1, keepdims=True)
    cent = o - mean
    var = jnp.mean(cent * cent, axis=-1, keepdims=True)
    return gm * cent * lax.rsqrt(var + 1e-5) + bt + ef


def _edge_body(ef_ref, g_ref, we_ref, wo_ref, b1_ref, bo_ref, gm_ref, bt_ref,
               out_ref):
    out_ref[...] = _edge_compute(
        ef_ref[...], g_ref[...], we_ref[...], wo_ref[...], b1_ref[...],
        bo_ref[...], gm_ref[...], bt_ref[...])


def _edge_body_alias(ef_ref, g_ref, we_ref, wo_ref, b1_ref, bo_ref, gm_ref,
                     bt_ref, prev_ref, out_ref):
    del prev_ref
    out_ref[...] = _edge_compute(
        ef_ref[...], g_ref[...], we_ref[...], wo_ref[...], b1_ref[...],
        bo_ref[...], gm_ref[...], bt_ref[...])


def _edge_mlp_half(efeat, gsum_h, W_e, W_out, b1, b_out, gamma, beta, half,
                   prev=None):
    vec = lambda: pl.BlockSpec((1, D), lambda i: (0, 0))
    off = half * _NBLK
    in_specs = [
        pl.BlockSpec((_EB, D), lambda i: (off + i, 0)),
        pl.BlockSpec((_EB, H), lambda i: (i, 0)),
        pl.BlockSpec((D, H), lambda i: (0, 0)),
        pl.BlockSpec((H, D), lambda i: (0, 0)),
        vec(), vec(), vec(), vec(),
    ]
    inputs = [efeat, gsum_h, W_e, W_out,
              b1.reshape(1, D), b_out.reshape(1, D),
              gamma.reshape(1, D), beta.reshape(1, D)]
    body = _edge_body
    io_alias = {}
    if prev is not None:
        in_specs.append(pl.BlockSpec(memory_space=pl.ANY))
        inputs.append(prev)
        body = _edge_body_alias
        io_alias = {8: 0}
    return pl.pallas_call(
        body,
        grid=(_NBLK,),
        in_specs=in_specs,
        out_specs=pl.BlockSpec((_EB, D), lambda i: (off + i, 0)),
        out_shape=jax.ShapeDtypeStruct((N_EDGES, D), jnp.float32),
        input_output_aliases=io_alias,
    )(*inputs)


def kernel(efeat, nfeat, edge_index, W_e, W_s, W_d, b1, W_out, b_out, gamma,
           beta):
    src = edge_index[0].astype(jnp.int32)
    dst = edge_index[1].astype(jnp.int32)
    fill = jnp.arange(_EH - _EHR, dtype=jnp.int32) % N_NODES
    halves = [
        (jnp.concatenate([src[h * _EHR:(h + 1) * _EHR], fill]),
         jnp.concatenate([dst[h * _EHR:(h + 1) * _EHR], fill]))
        for h in range(_NHALF)
    ]

    ps, pd = _project_nodes(nfeat, W_s, W_d)
    g0 = _gather_sum_half(halves[0][0], halves[0][1], ps, pd)
    g1 = _gather_sum_half(halves[1][0], halves[1][1], ps, pd)
    out = _edge_mlp_half(efeat, g0, W_e, W_out, b1, b_out, gamma, beta, 0)
    out = _edge_mlp_half(efeat, g1, W_e, W_out, b1, b_out, gamma, beta, 1,
                         prev=out)
    return (out, nfeat)


# speedup vs baseline: 1.2633x; 1.0162x over previous
import jax
import jax.numpy as jnp
from jax import lax
from jax.experimental import pallas as pl
from jax.experimental.pallas import tpu as pltpu
from jax.experimental.pallas import tpu_sc as plsc

N_NODES = 10000
N_EDGES = 320000
D = 128
H = 128


_NB = 2000


def _proj_body(nf_ref, ws_ref, wd_ref, ps_ref, pd_ref):
    x = nf_ref[...]
    ps_ref[...] = jnp.dot(x, ws_ref[...], preferred_element_type=jnp.float32)
    pd_ref[...] = jnp.dot(x, wd_ref[...], preferred_element_type=jnp.float32)


def _project_nodes(nfeat, W_s, W_d):
    return pl.pallas_call(
        _proj_body,
        grid=(N_NODES // _NB,),
        in_specs=[
            pl.BlockSpec((_NB, D), lambda i: (i, 0)),
            pl.BlockSpec((D, H), lambda i: (0, 0)),
            pl.BlockSpec((D, H), lambda i: (0, 0)),
        ],
        out_specs=[
            pl.BlockSpec((_NB, H), lambda i: (i, 0)),
            pl.BlockSpec((_NB, H), lambda i: (i, 0)),
        ],
        out_shape=[
            jax.ShapeDtypeStruct((N_NODES, H), jnp.float32),
            jax.ShapeDtypeStruct((N_NODES, H), jnp.float32),
        ],
    )(nfeat, W_s, W_d)


_NC = 2
_NS = 16
_NW = _NC * _NS
_C = 128
_NHALF = 4
_TPW = 20
_EPW = _TPW * _C
_EH = _NW * _EPW
_EHR = N_EDGES // _NHALF


def _gather_body(src_hbm, dst_hbm, ps_hbm, pd_hbm, out_hbm,
                 isrc0, idst0, ra0, rb0,
                 isrc1, idst1, ra1, rb1,
                 gsem0, gsem1):
    wid = lax.axis_index("s") * _NC + lax.axis_index("c")
    base = wid * _EPW
    slots = ((isrc0, idst0, ra0, rb0, gsem0),
             (isrc1, idst1, ra1, rb1, gsem1))

    def g_start(k, s):
        isrc, idst, ra, rb, gsem = slots[s]
        off = base + k * _C
        pltpu.sync_copy(src_hbm.at[pl.ds(off, _C)], isrc)
        pltpu.sync_copy(dst_hbm.at[pl.ds(off, _C)], idst)
        pltpu.async_copy(ps_hbm.at[isrc], ra, gsem)
        pltpu.async_copy(pd_hbm.at[idst], rb, gsem)

    def g_wait(s):
        isrc, idst, ra, rb, gsem = slots[s]
        pltpu.make_async_copy(ps_hbm.at[isrc], ra, gsem).wait()
        pltpu.make_async_copy(pd_hbm.at[idst], rb, gsem).wait()

    def add(s):
        _, _, ra, rb, _ = slots[s]

        def add_row(e, c2):
            for j in range(H // 16):
                sl = pl.ds(j * 16, 16)
                ra[e, sl] = ra[e, sl] + rb[e, sl]
            return c2

        lax.fori_loop(0, _C, add_row, 0)

    def wb_sync(k, s):
        _, _, ra, _, _ = slots[s]
        off = base + k * _C
        pltpu.sync_copy(ra, out_hbm.at[pl.ds(off, _C)])

    g_start(0, 0)

    def step(k2, carry):
        a = 2 * k2
        g_start(a + 1, 1)
        g_wait(0)
        add(0)
        wb_sync(a, 0)

        @pl.when(k2 < _TPW // 2 - 1)
        def _():
            g_start(a + 2, 0)

        g_wait(1)
        add(1)
        wb_sync(a + 1, 1)
        return carry

    lax.fori_loop(0, _TPW // 2, step, 0)


def _gather_sum_half(src_h, dst_h, ps, pd):
    mesh = plsc.VectorSubcoreMesh(core_axis_name="c", subcore_axis_name="s")
    f = pl.kernel(
        _gather_body,
        mesh=mesh,
        out_type=jax.ShapeDtypeStruct((_EH, H), jnp.float32),
        scratch_types=[
            pltpu.VMEM((_C,), jnp.int32),
            pltpu.VMEM((_C,), jnp.int32),
            pltpu.VMEM((_C, H), jnp.float32),
            pltpu.VMEM((_C, H), jnp.float32),
            pltpu.VMEM((_C,), jnp.int32),
            pltpu.VMEM((_C,), jnp.int32),
            pltpu.VMEM((_C, H), jnp.float32),
            pltpu.VMEM((_C, H), jnp.float32),
            pltpu.SemaphoreType.DMA,
            pltpu.SemaphoreType.DMA,
        ],
    )
    return f(src_h, dst_h, ps, pd)


_EB = 8000
_NBLK = (N_EDGES // _NHALF) // _EB


def _edge_compute(ef, g, we, wo, b1, bo, gm, bt):
    pre = jnp.dot(ef, we, preferred_element_type=jnp.float32) + g + b1
    h = pre * (1.0 / (1.0 + jnp.exp(-pre)))
    o = jnp.dot(h, wo, preferred_element_type=jnp.float32) + bo
    mean = jnp.mean(o, axis=-1, keepdims=True)
    cent = o - mean
    var = jnp.mean(cent * cent, axis=-1, keepdims=True)
    return gm * cent * lax.rsqrt(var + 1e-5) + bt + ef


def _edge_body(ef_ref, g_ref, we_ref, wo_ref, b1_ref, bo_ref, gm_ref, bt_ref,
               out_ref):
    out_ref[...] = _edge_compute(
        ef_ref[...], g_ref[...], we_ref[...], wo_ref[...], b1_ref[...],
        bo_ref[...], gm_ref[...], bt_ref[...])


def _edge_body_alias(ef_ref, g_ref, we_ref, wo_ref, b1_ref, bo_ref, gm_ref,
                     bt_ref, prev_ref, out_ref):
    del prev_ref
    out_ref[...] = _edge_compute(
        ef_ref[...], g_ref[...], we_ref[...], wo_ref[...], b1_ref[...],
        bo_ref[...], gm_ref[...], bt_ref[...])


def _edge_mlp_half(efeat, gsum_h, W_e, W_out, b1, b_out, gamma, beta, half,
                   prev=None):
    vec = lambda: pl.BlockSpec((1, D), lambda i: (0, 0))
    off = half * _NBLK
    in_specs = [
        pl.BlockSpec((_EB, D), lambda i: (off + i, 0)),
        pl.BlockSpec((_EB, H), lambda i: (i, 0)),
        pl.BlockSpec((D, H), lambda i: (0, 0)),
        pl.BlockSpec((H, D), lambda i: (0, 0)),
        vec(), vec(), vec(), vec(),
    ]
    inputs = [efeat, gsum_h, W_e, W_out,
              b1.reshape(1, D), b_out.reshape(1, D),
              gamma.reshape(1, D), beta.reshape(1, D)]
    body = _edge_body
    io_alias = {}
    if prev is not None:
        in_specs.append(pl.BlockSpec(memory_space=pl.ANY))
        inputs.append(prev)
        body = _edge_body_alias
        io_alias = {8: 0}
    return pl.pallas_call(
        body,
        grid=(_NBLK,),
        in_specs=in_specs,
        out_specs=pl.BlockSpec((_EB, D), lambda i: (off + i, 0)),
        out_shape=jax.ShapeDtypeStruct((N_EDGES, D), jnp.float32),
        input_output_aliases=io_alias,
    )(*inputs)


def kernel(efeat, nfeat, edge_index, W_e, W_s, W_d, b1, W_out, b_out, gamma,
           beta):
    src = edge_index[0].astype(jnp.int32)
    dst = edge_index[1].astype(jnp.int32)
    fill = jnp.arange(_EH - _EHR, dtype=jnp.int32) % N_NODES
    halves = [
        (jnp.concatenate([src[h * _EHR:(h + 1) * _EHR], fill]),
         jnp.concatenate([dst[h * _EHR:(h + 1) * _EHR], fill]))
        for h in range(_NHALF)
    ]

    ps, pd = _project_nodes(nfeat, W_s, W_d)
    gs = [_gather_sum_half(s_h, d_h, ps, pd) for s_h, d_h in halves]
    out = None
    for h in range(_NHALF):
        out = _edge_mlp_half(efeat, gs[h], W_e, W_out, b1, b_out, gamma,
                             beta, h, prev=out)
    return (out, nfeat)
